# Initial kernel scaffold; baseline (speedup 1.0000x reference)
import jax
import jax.numpy as jnp
from jax.experimental import pallas as pl


def kernel(inputs, true_duration, true_pitch, true_energy, mel_max_len, params):
    raise NotImplementedError("write your pallas kernel here")


# speedup vs baseline: 3.6588x; 3.6588x over previous
import functools

import jax
import jax.numpy as jnp
from jax import lax
from jax.experimental import pallas as pl
from jax.experimental.pallas import tpu as pltpu
from jax.experimental.pallas import tpu_sc as plsc

F32 = jnp.float32

_B, _S, _C = 8, 512, 256
_MEL = 4096
_NBINS = 256
_ROWS = _B * _MEL
_NC, _NS = 2, 16
_NW = _NC * _NS
_RPW = _ROWS // _NW
_CH = 128
_NCHK = _RPW // _CH


def _ln(h, g, be):
    m = jnp.mean(h, axis=-1, keepdims=True)
    c = h - m
    v = jnp.mean(c * c, axis=-1, keepdims=True)
    return c * lax.rsqrt(v + 1e-5) * g + be


def _pred_rows(x, wm1, wz, wp1, b1, g1, be1, w2m, w2z, w2p, b2, g2, be2, wl,
               bl):
    z = jnp.zeros((1, x.shape[1]), F32)
    p = jnp.dot(x, wm1, preferred_element_type=F32)
    q = jnp.dot(x, wz, preferred_element_type=F32)
    r = jnp.dot(x, wp1, preferred_element_type=F32)
    h = q + jnp.concatenate([z, p[:-1]], 0) + jnp.concatenate([r[1:], z], 0)
    h = jnp.maximum(h + b1, 0.0)
    h = _ln(h, g1, be1)
    p = jnp.dot(h, w2m, preferred_element_type=F32)
    q = jnp.dot(h, w2z, preferred_element_type=F32)
    r = jnp.dot(h, w2p, preferred_element_type=F32)
    h = q + jnp.concatenate([z, p[:-1]], 0) + jnp.concatenate([r[1:], z], 0)
    h = jnp.maximum(h + b2, 0.0)
    h = _ln(h, g2, be2)
    return jnp.sum(h * wl, axis=-1) + bl


def _k1_body(x_ref, dur_ref, tp_ref, te_ref, qp_ref, qe_ref, *rest):
    wrefs = rest[:14]
    dout, sgo, pio, eio = rest[14:]
    b = pl.program_id(0)
    x = x_ref[0]
    w = [wrefs[i][...] for i in range(13)] + [wrefs[13][0]]
    dout[0, 0, :] = _pred_rows(x, *w)

    d = dur_ref[0, 0, :].astype(F32)
    ii = lax.broadcasted_iota(jnp.int32, (_S, _S), 0)
    jj = lax.broadcasted_iota(jnp.int32, (_S, _S), 1)
    tri = (ii <= jj).astype(F32)
    cum = jnp.dot(d.reshape(1, _S), tri,
                  preferred_element_type=F32).astype(jnp.int32)
    for ci in range(4):
        t = lax.broadcasted_iota(jnp.int32, (_MEL // 4, _S), 0) + (
            ci * (_MEL // 4))
        cnt = jnp.sum((cum <= t).astype(F32), axis=1).astype(jnp.int32)
        sgo[0, 0, pl.ds(ci * (_MEL // 4), _MEL // 4)] = cnt + b * (_S + 1)

    xp = tp_ref[0, 0, :]
    pio[0, 0, :] = jnp.sum((qp_ref[...] < xp[:, None]).astype(F32),
                           axis=1).astype(jnp.int32)
    xe = te_ref[0, 0, :]
    eio[0, 0, :] = jnp.sum((qe_ref[...] < xe[:, None]).astype(F32),
                           axis=1).astype(jnp.int32)


def _full_spec(shape):
    nd = len(shape)
    return pl.BlockSpec(shape, lambda b, _n=nd: (0,) * _n)


def _k1(x, dur3, tp3, te3, qp, qe, dw):
    wspecs = [_full_spec(w.shape) for w in dw[:13]]
    wspecs.append(pl.BlockSpec(memory_space=pltpu.SMEM))
    return pl.pallas_call(
        _k1_body,
        grid=(_B,),
        in_specs=[
            pl.BlockSpec((1, _S, _C), lambda b: (b, 0, 0)),
            pl.BlockSpec((1, 1, _S), lambda b: (b, 0, 0)),
            pl.BlockSpec((1, 1, _MEL), lambda b: (b, 0, 0)),
            pl.BlockSpec((1, 1, _MEL), lambda b: (b, 0, 0)),
            _full_spec((_NBINS,)),
            _full_spec((_NBINS,)),
        ] + wspecs,
        out_specs=[
            pl.BlockSpec((1, 1, _S), lambda b: (b, 0, 0)),
            pl.BlockSpec((1, 1, _MEL), lambda b: (b, 0, 0)),
            pl.BlockSpec((1, 1, _MEL), lambda b: (b, 0, 0)),
            pl.BlockSpec((1, 1, _MEL), lambda b: (b, 0, 0)),
        ],
        out_shape=[
            jax.ShapeDtypeStruct((_B, 1, _S), F32),
            jax.ShapeDtypeStruct((_B, 1, _MEL), jnp.int32),
            jax.ShapeDtypeStruct((_B, 1, _MEL), jnp.int32),
            jax.ShapeDtypeStruct((_B, 1, _MEL), jnp.int32),
        ],
    )(x, dur3, tp3, te3, qp, qe, *dw)


def _sc_gather(xext, pemb, eemb, sidx, pidx, eidx):
    mesh = plsc.VectorSubcoreMesh(core_axis_name="c", subcore_axis_name="s")

    @functools.partial(
        pl.kernel,
        out_type=[jax.ShapeDtypeStruct((_ROWS, _C), F32)] * 3,
        mesh=mesh,
        scratch_types=[
            pltpu.VMEM((_NCHK, _CH), jnp.int32),
            pltpu.VMEM((_NCHK, _CH), jnp.int32),
            pltpu.VMEM((_NCHK, _CH), jnp.int32),
            pltpu.VMEM((_CH, _C), F32),
            pltpu.VMEM((_CH, _C), F32),
            pltpu.VMEM((_CH, _C), F32),
            pltpu.SemaphoreType.DMA,
            pltpu.SemaphoreType.DMA,
            pltpu.SemaphoreType.DMA,
        ],
    )
    def k(xe_h, pe_h, ee_h, si_h, pi_h, ei_h, oa, op_, oe, si_v, pi_v, ei_v,
          ba, bp, be, sa, sp, se):
        wid = lax.axis_index("s") * _NC + lax.axis_index("c")
        base = wid * _RPW
        pltpu.sync_copy(si_h.at[pl.ds(wid * _NCHK, _NCHK)], si_v)
        pltpu.sync_copy(pi_h.at[pl.ds(wid * _NCHK, _NCHK)], pi_v)
        pltpu.sync_copy(ei_h.at[pl.ds(wid * _NCHK, _NCHK)], ei_v)
        for j in range(_NCHK):
            ga = pltpu.async_copy(xe_h.at[si_v.at[j]], ba, sa)
            gp = pltpu.async_copy(pe_h.at[pi_v.at[j]], bp, sp)
            ge = pltpu.async_copy(ee_h.at[ei_v.at[j]], be, se)
            row0 = base + j * _CH
            ga.wait()
            pltpu.sync_copy(ba, oa.at[pl.ds(row0, _CH)])
            gp.wait()
            pltpu.sync_copy(bp, op_.at[pl.ds(row0, _CH)])
            ge.wait()
            pltpu.sync_copy(be, oe.at[pl.ds(row0, _CH)])

    return k(xext, pemb, eemb, sidx, pidx, eidx)


def _k3_body(a_ref, p_ref, e_ref, *rest):
    wrefs = rest[:28]
    out_ref, pit_ref, en_ref = rest[28:]
    pwl = [wrefs[i][...] for i in range(13)] + [wrefs[13][0]]
    ewl = [wrefs[14 + i][...] for i in range(13)] + [wrefs[27][0]]
    a = a_ref[0]
    pit_ref[0, 0, :] = _pred_rows(a, *pwl)
    ap = a + p_ref[0]
    en_ref[0, 0, :] = _pred_rows(ap, *ewl)
    out_ref[0] = ap + e_ref[0]


def _k3(a, p, e, pw, ew):
    wspecs = []
    for ws in (pw, ew):
        wspecs += [_full_spec(w.shape) for w in ws[:13]]
        wspecs.append(pl.BlockSpec(memory_space=pltpu.SMEM))
    return pl.pallas_call(
        _k3_body,
        grid=(_B,),
        in_specs=[
            pl.BlockSpec((1, _MEL, _C), lambda b: (b, 0, 0)),
            pl.BlockSpec((1, _MEL, _C), lambda b: (b, 0, 0)),
            pl.BlockSpec((1, _MEL, _C), lambda b: (b, 0, 0)),
        ] + wspecs,
        out_specs=[
            pl.BlockSpec((1, _MEL, _C), lambda b: (b, 0, 0)),
            pl.BlockSpec((1, 1, _MEL), lambda b: (b, 0, 0)),
            pl.BlockSpec((1, 1, _MEL), lambda b: (b, 0, 0)),
        ],
        out_shape=[
            jax.ShapeDtypeStruct((_B, _MEL, _C), F32),
            jax.ShapeDtypeStruct((_B, 1, _MEL), F32),
            jax.ShapeDtypeStruct((_B, 1, _MEL), F32),
        ],
    )(a, p, e, *pw, *ew)


def _wsplit(p):
    return [p["w1"][:, :, 0].T, p["w1"][:, :, 1].T, p["w1"][:, :, 2].T,
            p["b1"], p["g1"], p["be1"],
            p["w2"][:, :, 0].T, p["w2"][:, :, 1].T, p["w2"][:, :, 2].T,
            p["b2"], p["g2"], p["be2"], p["wl"][:, 0], p["bl"]]


def kernel(inputs, true_duration, true_pitch, true_energy, mel_max_len,
           params):
    qp = jnp.concatenate(
        [jnp.linspace(-3.0, 200.0, _NBINS - 1),
         jnp.full((1,), jnp.inf)]).astype(F32)
    qe = jnp.concatenate(
        [jnp.linspace(-2.0, 200.0, _NBINS - 1),
         jnp.full((1,), jnp.inf)]).astype(F32)
    dw = _wsplit(params["dur"])
    pw = _wsplit(params["pitch_pred"])
    ew = _wsplit(params["energy_pred"])
    dur3 = true_duration.reshape(_B, 1, _S)
    tp3 = true_pitch.reshape(_B, 1, _MEL)
    te3 = true_energy.reshape(_B, 1, _MEL)

    durations3, sg3, pi3, ei3 = _k1(inputs, dur3, tp3, te3, qp, qe, dw)

    xext = jnp.concatenate(
        [inputs, jnp.zeros((_B, 1, _C), F32)], axis=1).reshape(-1, _C)
    a, p, e = _sc_gather(xext, params["pitch_emb"], params["energy_emb"],
                         sg3.reshape(-1, _CH), pi3.reshape(-1, _CH),
                         ei3.reshape(-1, _CH))

    out, pit3, en3 = _k3(a.reshape(_B, _MEL, _C), p.reshape(_B, _MEL, _C),
                         e.reshape(_B, _MEL, _C), pw, ew)
    return (out, durations3.reshape(_B, _S), pit3.reshape(_B, _MEL),
            en3.reshape(_B, _MEL))


# speedup vs baseline: 11.2973x; 3.0877x over previous
import functools

import jax
import jax.numpy as jnp
from jax import lax
from jax.experimental import pallas as pl
from jax.experimental.pallas import tpu as pltpu
from jax.experimental.pallas import tpu_sc as plsc

F32 = jnp.float32

_B, _S, _C = 8, 512, 256
_MEL = 4096
_NBINS = 256
_ROWS = _B * _MEL
_NC, _NS = 2, 16
_NW = _NC * _NS
_RPW = _ROWS // _NW
_CH = 128
_NCHK = _RPW // _CH


def _ln(h, g, be):
    m = jnp.mean(h, axis=-1, keepdims=True)
    c = h - m
    v = jnp.mean(c * c, axis=-1, keepdims=True)
    return c * lax.rsqrt(v + 1e-5) * g + be


def _pred_rows(x, wm1, wz, wp1, b1, g1, be1, w2m, w2z, w2p, b2, g2, be2, wl,
               bl):
    z = jnp.zeros((1, x.shape[1]), F32)
    p = jnp.dot(x, wm1, preferred_element_type=F32)
    q = jnp.dot(x, wz, preferred_element_type=F32)
    r = jnp.dot(x, wp1, preferred_element_type=F32)
    h = q + jnp.concatenate([z, p[:-1]], 0) + jnp.concatenate([r[1:], z], 0)
    h = jnp.maximum(h + b1, 0.0)
    h = _ln(h, g1, be1)
    p = jnp.dot(h, w2m, preferred_element_type=F32)
    q = jnp.dot(h, w2z, preferred_element_type=F32)
    r = jnp.dot(h, w2p, preferred_element_type=F32)
    h = q + jnp.concatenate([z, p[:-1]], 0) + jnp.concatenate([r[1:], z], 0)
    h = jnp.maximum(h + b2, 0.0)
    h = _ln(h, g2, be2)
    return jnp.sum(h * wl, axis=-1) + bl


def _k1_body(x_ref, dur_ref, tp_ref, te_ref, qp_ref, qe_ref, *rest):
    wrefs = rest[:14]
    dout, sgo, pio, eio = rest[14:]
    b = pl.program_id(0)
    x = x_ref[0]
    w = [wrefs[i][...] for i in range(13)] + [wrefs[13][0]]
    dout[0, 0, :] = _pred_rows(x, *w)

    d = dur_ref[0, 0, :].astype(F32)
    ii = lax.broadcasted_iota(jnp.int32, (_S, _S), 0)
    jj = lax.broadcasted_iota(jnp.int32, (_S, _S), 1)
    tri = (ii <= jj).astype(F32)
    cum = jnp.dot(d.reshape(1, _S), tri,
                  preferred_element_type=F32).astype(jnp.int32)
    for ci in range(4):
        t = lax.broadcasted_iota(jnp.int32, (_MEL // 4, _S), 0) + (
            ci * (_MEL // 4))
        cnt = jnp.sum((cum <= t).astype(F32), axis=1).astype(jnp.int32)
        sgo[0, 0, pl.ds(ci * (_MEL // 4), _MEL // 4)] = cnt + b * (_S + 1)

    xp = tp_ref[0, 0, :]
    pio[0, 0, :] = jnp.sum((qp_ref[...] < xp[:, None]).astype(F32),
                           axis=1).astype(jnp.int32)
    xe = te_ref[0, 0, :]
    eio[0, 0, :] = jnp.sum((qe_ref[...] < xe[:, None]).astype(F32),
                           axis=1).astype(jnp.int32)


def _full_spec(shape):
    nd = len(shape)
    return pl.BlockSpec(shape, lambda b, _n=nd: (0,) * _n)


def _k1(x, dur3, tp3, te3, qp, qe, dw):
    wspecs = [_full_spec(w.shape) for w in dw[:13]]
    wspecs.append(pl.BlockSpec(memory_space=pltpu.SMEM))
    return pl.pallas_call(
        _k1_body,
        grid=(_B,),
        in_specs=[
            pl.BlockSpec((1, _S, _C), lambda b: (b, 0, 0)),
            pl.BlockSpec((1, 1, _S), lambda b: (b, 0, 0)),
            pl.BlockSpec((1, 1, _MEL), lambda b: (b, 0, 0)),
            pl.BlockSpec((1, 1, _MEL), lambda b: (b, 0, 0)),
            _full_spec((_NBINS,)),
            _full_spec((_NBINS,)),
        ] + wspecs,
        out_specs=[
            pl.BlockSpec((1, 1, _S), lambda b: (b, 0, 0)),
            pl.BlockSpec((1, 1, _MEL), lambda b: (b, 0, 0)),
            pl.BlockSpec((1, 1, _MEL), lambda b: (b, 0, 0)),
            pl.BlockSpec((1, 1, _MEL), lambda b: (b, 0, 0)),
        ],
        out_shape=[
            jax.ShapeDtypeStruct((_B, 1, _S), F32),
            jax.ShapeDtypeStruct((_B, 1, _MEL), jnp.int32),
            jax.ShapeDtypeStruct((_B, 1, _MEL), jnp.int32),
            jax.ShapeDtypeStruct((_B, 1, _MEL), jnp.int32),
        ],
    )(x, dur3, tp3, te3, qp, qe, *dw)


def _sc_gather(xext, sidx):
    mesh = plsc.VectorSubcoreMesh(core_axis_name="c", subcore_axis_name="s")

    @functools.partial(
        pl.kernel,
        out_type=jax.ShapeDtypeStruct((_ROWS, _C), F32),
        mesh=mesh,
        scratch_types=[
            pltpu.VMEM((_NCHK, _CH), jnp.int32),
            pltpu.VMEM((2, _CH, _C), F32),
            pltpu.SemaphoreType.DMA,
            pltpu.SemaphoreType.DMA,
            pltpu.SemaphoreType.DMA,
            pltpu.SemaphoreType.DMA,
        ],
    )
    def k(xe_h, si_h, oa, si_v, bufs, sg0, sg1, ss0, ss1):
        wid = lax.axis_index("s") * _NC + lax.axis_index("c")
        base = wid * _RPW
        sg = (sg0, sg1)
        ss = (ss0, ss1)
        pltpu.sync_copy(si_h.at[pl.ds(wid * _NCHK, _NCHK)], si_v)
        gat = [None] * _NCHK
        sct = [None] * _NCHK
        for j in range(_NCHK):
            s = j % 2
            if j >= 2:
                sct[j - 2].wait()
            gat[j] = pltpu.async_copy(xe_h.at[si_v.at[j]], bufs.at[s], sg[s])
            if j >= 1:
                gat[j - 1].wait()
                sp = (j - 1) % 2
                sct[j - 1] = pltpu.async_copy(
                    bufs.at[sp], oa.at[pl.ds(base + (j - 1) * _CH, _CH)],
                    ss[sp])
        gat[_NCHK - 1].wait()
        sct[_NCHK - 2].wait()
        last = (_NCHK - 1) % 2
        sct[_NCHK - 1] = pltpu.async_copy(
            bufs.at[last], oa.at[pl.ds(base + (_NCHK - 1) * _CH, _CH)],
            ss[last])
        sct[_NCHK - 1].wait()

    return k(xext, sidx)


def _onehot_rows(idx, emb):
    oh = (idx[:, None] == lax.broadcasted_iota(jnp.int32, (_MEL, _NBINS),
                                               1)).astype(F32)
    return jnp.dot(oh, emb, preferred_element_type=F32)


def _k3_body(a_ref, pi_ref, ei_ref, pemb_ref, eemb_ref, *rest):
    wrefs = rest[:28]
    out_ref, pit_ref, en_ref = rest[28:]
    pwl = [wrefs[i][...] for i in range(13)] + [wrefs[13][0]]
    ewl = [wrefs[14 + i][...] for i in range(13)] + [wrefs[27][0]]
    a = a_ref[0]
    pit_ref[0, 0, :] = _pred_rows(a, *pwl)
    ap = a + _onehot_rows(pi_ref[0, 0, :], pemb_ref[...])
    en_ref[0, 0, :] = _pred_rows(ap, *ewl)
    out_ref[0] = ap + _onehot_rows(ei_ref[0, 0, :], eemb_ref[...])


def _k3(a, pi3, ei3, pemb, eemb, pw, ew):
    wspecs = []
    for ws in (pw, ew):
        wspecs += [_full_spec(w.shape) for w in ws[:13]]
        wspecs.append(pl.BlockSpec(memory_space=pltpu.SMEM))
    return pl.pallas_call(
        _k3_body,
        grid=(_B,),
        in_specs=[
            pl.BlockSpec((1, _MEL, _C), lambda b: (b, 0, 0)),
            pl.BlockSpec((1, 1, _MEL), lambda b: (b, 0, 0)),
            pl.BlockSpec((1, 1, _MEL), lambda b: (b, 0, 0)),
            _full_spec((_NBINS, _C)),
            _full_spec((_NBINS, _C)),
        ] + wspecs,
        out_specs=[
            pl.BlockSpec((1, _MEL, _C), lambda b: (b, 0, 0)),
            pl.BlockSpec((1, 1, _MEL), lambda b: (b, 0, 0)),
            pl.BlockSpec((1, 1, _MEL), lambda b: (b, 0, 0)),
        ],
        out_shape=[
            jax.ShapeDtypeStruct((_B, _MEL, _C), F32),
            jax.ShapeDtypeStruct((_B, 1, _MEL), F32),
            jax.ShapeDtypeStruct((_B, 1, _MEL), F32),
        ],
    )(a, pi3, ei3, pemb, eemb, *pw, *ew)


def _wsplit(p):
    return [p["w1"][:, :, 0].T, p["w1"][:, :, 1].T, p["w1"][:, :, 2].T,
            p["b1"], p["g1"], p["be1"],
            p["w2"][:, :, 0].T, p["w2"][:, :, 1].T, p["w2"][:, :, 2].T,
            p["b2"], p["g2"], p["be2"], p["wl"][:, 0], p["bl"]]


def kernel(inputs, true_duration, true_pitch, true_energy, mel_max_len,
           params):
    qp = jnp.concatenate(
        [jnp.linspace(-3.0, 200.0, _NBINS - 1),
         jnp.full((1,), jnp.inf)]).astype(F32)
    qe = jnp.concatenate(
        [jnp.linspace(-2.0, 200.0, _NBINS - 1),
         jnp.full((1,), jnp.inf)]).astype(F32)
    dw = _wsplit(params["dur"])
    pw = _wsplit(params["pitch_pred"])
    ew = _wsplit(params["energy_pred"])
    dur3 = true_duration.reshape(_B, 1, _S)
    tp3 = true_pitch.reshape(_B, 1, _MEL)
    te3 = true_energy.reshape(_B, 1, _MEL)

    durations3, sg3, pi3, ei3 = _k1(inputs, dur3, tp3, te3, qp, qe, dw)

    xext = jnp.concatenate(
        [inputs, jnp.zeros((_B, 1, _C), F32)], axis=1).reshape(-1, _C)
    a = _sc_gather(xext, sg3.reshape(-1, _CH))

    out, pit3, en3 = _k3(a.reshape(_B, _MEL, _C), pi3, ei3,
                         params["pitch_emb"], params["energy_emb"], pw, ew)
    return (out, durations3.reshape(_B, _S), pit3.reshape(_B, _MEL),
            en3.reshape(_B, _MEL))


# speedup vs baseline: 12.7767x; 1.1310x over previous
import functools

import jax
import jax.numpy as jnp
from jax import lax
from jax.experimental import pallas as pl
from jax.experimental.pallas import tpu as pltpu
from jax.experimental.pallas import tpu_sc as plsc

F32 = jnp.float32

_B, _S, _C = 8, 512, 256
_MEL = 4096
_NBINS = 256
_ROWS = _B * _MEL
_NC, _NS = 2, 16
_NW = _NC * _NS
_RPW = _ROWS // _NW
_CH = 128
_NCHK = _RPW // _CH


def _ln(h, g, be):
    m = jnp.mean(h, axis=-1, keepdims=True)
    c = h - m
    v = jnp.mean(c * c, axis=-1, keepdims=True)
    return c * lax.rsqrt(v + 1e-5) * g + be


def _pred_rows(x, wm1, wz, wp1, b1, g1, be1, w2m, w2z, w2p, b2, wg, sc):
    z = jnp.zeros((1, x.shape[1]), F32)
    p = jnp.dot(x, wm1, preferred_element_type=F32)
    q = jnp.dot(x, wz, preferred_element_type=F32)
    r = jnp.dot(x, wp1, preferred_element_type=F32)
    h = q + jnp.concatenate([z, p[:-1]], 0) + jnp.concatenate([r[1:], z], 0)
    h = jnp.maximum(h + b1, 0.0)
    h = _ln(h, g1, be1)
    p = jnp.dot(h, w2m, preferred_element_type=F32)
    q = jnp.dot(h, w2z, preferred_element_type=F32)
    r = jnp.dot(h, w2p, preferred_element_type=F32)
    h = q + jnp.concatenate([z, p[:-1]], 0) + jnp.concatenate([r[1:], z], 0)
    h = jnp.maximum(h + b2, 0.0)
    m = jnp.mean(h, axis=-1, keepdims=True)
    v = jnp.mean(h * h, axis=-1, keepdims=True) - m * m
    num = jnp.sum(h * wg, axis=-1, keepdims=True)
    return ((num - m * sc[0]) * lax.rsqrt(v + 1e-5))[:, 0] + sc[1]


def _k1_body(dur_ref, sgo):
    b = pl.program_id(0)
    d = dur_ref[0, 0, :].astype(F32)
    ii = lax.broadcasted_iota(jnp.int32, (_S, _S), 0)
    jj = lax.broadcasted_iota(jnp.int32, (_S, _S), 1)
    tri = (ii <= jj).astype(F32)
    cum = jnp.dot(d.reshape(1, _S), tri,
                  preferred_element_type=F32).astype(jnp.int32)
    for ci in range(4):
        t = lax.broadcasted_iota(jnp.int32, (_MEL // 4, _S), 0) + (
            ci * (_MEL // 4))
        cnt = jnp.sum((cum <= t).astype(F32), axis=1).astype(jnp.int32)
        sgo[0, 0, pl.ds(ci * (_MEL // 4), _MEL // 4)] = cnt + b * (_S + 1)


def _full_spec(shape):
    nd = len(shape)
    return pl.BlockSpec(shape, lambda b, _n=nd: (0,) * _n)


def _k1(dur3):
    return pl.pallas_call(
        _k1_body,
        grid=(_B,),
        in_specs=[pl.BlockSpec((1, 1, _S), lambda b: (b, 0, 0))],
        out_specs=pl.BlockSpec((1, 1, _MEL), lambda b: (b, 0, 0)),
        out_shape=jax.ShapeDtypeStruct((_B, 1, _MEL), jnp.int32),
    )(dur3)


def _k2_body(x_ref, tp_ref, te_ref, qp_ref, qe_ref, *rest):
    wrefs = rest[:12]
    dout, pio, eio = rest[12:]
    x = x_ref[0]
    w = [wrefs[i][...] for i in range(11)] + [wrefs[11]]
    dout[0, 0, :] = _pred_rows(x, *w)
    xp = tp_ref[0, 0, :]
    pio[0, 0, :] = jnp.sum((qp_ref[...] < xp[:, None]).astype(F32),
                           axis=1).astype(jnp.int32)
    xe = te_ref[0, 0, :]
    eio[0, 0, :] = jnp.sum((qe_ref[...] < xe[:, None]).astype(F32),
                           axis=1).astype(jnp.int32)


def _k2(x, tp3, te3, qp, qe, dw):
    wspecs = [_full_spec(w.shape) for w in dw[:11]]
    wspecs.append(pl.BlockSpec(memory_space=pltpu.SMEM))
    return pl.pallas_call(
        _k2_body,
        grid=(_B,),
        in_specs=[
            pl.BlockSpec((1, _S, _C), lambda b: (b, 0, 0)),
            pl.BlockSpec((1, 1, _MEL), lambda b: (b, 0, 0)),
            pl.BlockSpec((1, 1, _MEL), lambda b: (b, 0, 0)),
            _full_spec((_NBINS,)),
            _full_spec((_NBINS,)),
        ] + wspecs,
        out_specs=[
            pl.BlockSpec((1, 1, _S), lambda b: (b, 0, 0)),
            pl.BlockSpec((1, 1, _MEL), lambda b: (b, 0, 0)),
            pl.BlockSpec((1, 1, _MEL), lambda b: (b, 0, 0)),
        ],
        out_shape=[
            jax.ShapeDtypeStruct((_B, 1, _S), F32),
            jax.ShapeDtypeStruct((_B, 1, _MEL), jnp.int32),
            jax.ShapeDtypeStruct((_B, 1, _MEL), jnp.int32),
        ],
    )(x, tp3, te3, qp, qe, *dw)


def _sc_gather(xext, sidx):
    mesh = plsc.VectorSubcoreMesh(core_axis_name="c", subcore_axis_name="s")

    @functools.partial(
        pl.kernel,
        out_type=jax.ShapeDtypeStruct((_ROWS, _C), F32),
        mesh=mesh,
        scratch_types=[
            pltpu.VMEM((_NCHK, _CH), jnp.int32),
            pltpu.VMEM((3, _CH, _C), F32),
            pltpu.SemaphoreType.DMA,
            pltpu.SemaphoreType.DMA,
            pltpu.SemaphoreType.DMA,
            pltpu.SemaphoreType.DMA,
            pltpu.SemaphoreType.DMA,
            pltpu.SemaphoreType.DMA,
        ],
    )
    def k(xe_h, si_h, oa, si_v, bufs, sg0, sg1, sg2, ss0, ss1, ss2):
        wid = lax.axis_index("s") * _NC + lax.axis_index("c")
        base = wid * _RPW
        sg = (sg0, sg1, sg2)
        ss = (ss0, ss1, ss2)
        pltpu.sync_copy(si_h.at[pl.ds(wid * _NCHK, _NCHK)], si_v)
        gat = [None] * _NCHK
        sct = [None] * _NCHK

        def _scatter(j):
            gat[j].wait()
            sct[j] = pltpu.async_copy(
                bufs.at[j % 3], oa.at[pl.ds(base + j * _CH, _CH)], ss[j % 3])

        for j in range(_NCHK):
            s = j % 3
            if j >= 3:
                sct[j - 3].wait()
            gat[j] = pltpu.async_copy(xe_h.at[si_v.at[j]], bufs.at[s], sg[s])
            if j >= 2:
                _scatter(j - 2)
        _scatter(_NCHK - 2)
        _scatter(_NCHK - 1)
        for j in range(_NCHK - 3, _NCHK):
            sct[j].wait()

    return k(xext, sidx)


def _onehot_rows(idx, emb):
    oh = (idx[:, None] == lax.broadcasted_iota(jnp.int32, (_MEL, _NBINS),
                                               1)).astype(F32)
    return jnp.dot(oh, emb, preferred_element_type=F32)


def _k3_body(a_ref, pi_ref, ei_ref, pemb_ref, eemb_ref, *rest):
    wrefs = rest[:24]
    out_ref, pit_ref, en_ref = rest[24:]
    pwl = [wrefs[i][...] for i in range(11)] + [wrefs[11]]
    ewl = [wrefs[12 + i][...] for i in range(11)] + [wrefs[23]]
    a = a_ref[0]
    pit_ref[0, 0, :] = _pred_rows(a, *pwl)
    ap = a + _onehot_rows(pi_ref[0, 0, :], pemb_ref[...])
    en_ref[0, 0, :] = _pred_rows(ap, *ewl)
    out_ref[0] = ap + _onehot_rows(ei_ref[0, 0, :], eemb_ref[...])


def _k3(a, pi3, ei3, pemb, eemb, pw, ew):
    wspecs = []
    for ws in (pw, ew):
        wspecs += [_full_spec(w.shape) for w in ws[:11]]
        wspecs.append(pl.BlockSpec(memory_space=pltpu.SMEM))
    return pl.pallas_call(
        _k3_body,
        grid=(_B,),
        in_specs=[
            pl.BlockSpec((1, _MEL, _C), lambda b: (b, 0, 0)),
            pl.BlockSpec((1, 1, _MEL), lambda b: (b, 0, 0)),
            pl.BlockSpec((1, 1, _MEL), lambda b: (b, 0, 0)),
            _full_spec((_NBINS, _C)),
            _full_spec((_NBINS, _C)),
        ] + wspecs,
        out_specs=[
            pl.BlockSpec((1, _MEL, _C), lambda b: (b, 0, 0)),
            pl.BlockSpec((1, 1, _MEL), lambda b: (b, 0, 0)),
            pl.BlockSpec((1, 1, _MEL), lambda b: (b, 0, 0)),
        ],
        out_shape=[
            jax.ShapeDtypeStruct((_B, _MEL, _C), F32),
            jax.ShapeDtypeStruct((_B, 1, _MEL), F32),
            jax.ShapeDtypeStruct((_B, 1, _MEL), F32),
        ],
    )(a, pi3, ei3, pemb, eemb, *pw, *ew)


def _wsplit(p):
    wl = p["wl"][:, 0]
    wg = p["g2"] * wl
    sc = jnp.stack([jnp.sum(wg), jnp.dot(p["be2"], wl) + p["bl"][0]])
    return [p["w1"][:, :, 0].T, p["w1"][:, :, 1].T, p["w1"][:, :, 2].T,
            p["b1"], p["g1"], p["be1"],
            p["w2"][:, :, 0].T, p["w2"][:, :, 1].T, p["w2"][:, :, 2].T,
            p["b2"], wg, sc]


def kernel(inputs, true_duration, true_pitch, true_energy, mel_max_len,
           params):
    qp = jnp.concatenate(
        [jnp.linspace(-3.0, 200.0, _NBINS - 1),
         jnp.full((1,), jnp.inf)]).astype(F32)
    qe = jnp.concatenate(
        [jnp.linspace(-2.0, 200.0, _NBINS - 1),
         jnp.full((1,), jnp.inf)]).astype(F32)
    dw = _wsplit(params["dur"])
    pw = _wsplit(params["pitch_pred"])
    ew = _wsplit(params["energy_pred"])
    dur3 = true_duration.reshape(_B, 1, _S)
    tp3 = true_pitch.reshape(_B, 1, _MEL)
    te3 = true_energy.reshape(_B, 1, _MEL)

    sg3 = _k1(dur3)

    xext = jnp.concatenate(
        [inputs, jnp.zeros((_B, 1, _C), F32)], axis=1).reshape(-1, _C)
    a = _sc_gather(xext, sg3.reshape(-1, _CH))

    durations3, pi3, ei3 = _k2(inputs, tp3, te3, qp, qe, dw)

    out, pit3, en3 = _k3(a.reshape(_B, _MEL, _C), pi3, ei3,
                         params["pitch_emb"], params["energy_emb"], pw, ew)
    return (out, durations3.reshape(_B, _S), pit3.reshape(_B, _MEL),
            en3.reshape(_B, _MEL))
